# baseline (speedup 1.0000x reference)
import functools

import jax
import jax.numpy as jnp
from jax import lax
from jax.experimental import pallas as pl
from jax.experimental.pallas import tpu as pltpu
import jax.experimental.pallas.tpu_sc as plsc

_GRAD_LIMIT = 30000.0
_EPS = 1e-08

_NUM_CORES = 2
_NUM_SUBCORES = 16
_NUM_TILES = _NUM_CORES * _NUM_SUBCORES

_CHUNK = 128
_CHUNKS_PER_SUPER = 16
_SUPER = _CHUNK * _CHUNKS_PER_SUPER
_NBUF = 3


def _sc_accumulate(row2d, col2d, px, py, f0, f1, f2, f3, zeros, npad,
                   pairs_per_tile):
  per_tile = pairs_per_tile * 2 * _SUPER
  node_chunk = npad // _NUM_SUBCORES

  mesh = plsc.VectorSubcoreMesh(core_axis_name="c", subcore_axis_name="s")

  tbl_t = [pltpu.VMEM_SHARED((npad,), jnp.float32) for _ in range(6)]
  acc_t = [pltpu.VMEM_SHARED((npad,), jnp.float32) for _ in range(11)]

  nblks = row2d.shape[0]

  @functools.partial(
      pl.kernel,
      out_type=jax.ShapeDtypeStruct((_NUM_CORES * 11 * npad,), jnp.float32),
      mesh=mesh,
      scratch_types=tbl_t + acc_t + [
          pltpu.VMEM((2, _CHUNKS_PER_SUPER, _CHUNK), jnp.int32),
          pltpu.VMEM((2, _CHUNKS_PER_SUPER, _CHUNK), jnp.int32),
          pltpu.VMEM((_NBUF * 12 * _CHUNK,), jnp.float32),
          pltpu.VMEM((_NBUF * 11 * _CHUNK,), jnp.float32),
          pltpu.SemaphoreType.DMA,
          pltpu.SemaphoreType.DMA,
          pltpu.SemaphoreType.DMA,
      ],
  )
  def kern(row_hbm, col_hbm, px_h, py_h, f0_h, f1_h, f2_h, f3_h, z_h,
           out_hbm, sx, sy, sf0, sf1, sf2, sf3,
           a0, a1, a2, a3, a4, a5, a6, a7, a8, a9, a10,
           ridx, cidx, g, o, semg, sems, semi):
    cid = lax.axis_index("c")
    sid = lax.axis_index("s")
    tbls = (sx, sy, sf0, sf1, sf2, sf3)
    tbl_srcs = (px_h, py_h, f0_h, f1_h, f2_h, f3_h)
    accs = (a0, a1, a2, a3, a4, a5, a6, a7, a8, a9, a10)

    nsl = pl.ds(sid * node_chunk, node_chunk)
    for dst, src in zip(tbls, tbl_srcs):
      pltpu.sync_copy(src.at[nsl], dst.at[nsl])
    for a in accs:
      pltpu.sync_copy(z_h.at[nsl], a.at[nsl])
    plsc.subcore_barrier()

    tile_base_blk = (cid * _NUM_SUBCORES + sid) * (per_tile // _CHUNK)

    def fire_idx(blk, b):
      return [
          pltpu.async_copy(
              row_hbm.at[pl.ds(blk, _CHUNKS_PER_SUPER)], ridx.at[b], semi),
          pltpu.async_copy(
              col_hbm.at[pl.ds(blk, _CHUNKS_PER_SUPER)], cidx.at[b], semi),
      ]

    def fire_gathers(b, j, buf):
      gb = buf * 12 * _CHUNK
      rj = ridx.at[b, j]
      cj = cidx.at[b, j]
      hs = []
      for t in range(6):
        hs.append(pltpu.async_copy(
            tbls[t].at[rj], g.at[pl.ds(gb + t * _CHUNK, _CHUNK)], semg))
        hs.append(pltpu.async_copy(
            tbls[t].at[cj], g.at[pl.ds(gb + (6 + t) * _CHUNK, _CHUNK)],
            semg))
      return hs

    def compute(buf):
      gb = buf * 12 * _CHUNK
      ob = buf * 11 * _CHUNK
      for i in range(_CHUNK // 16):
        sl = lambda t: pl.ds(gb + t * _CHUNK + i * 16, 16)
        osl = lambda t: pl.ds(ob + t * _CHUNK + i * 16, 16)
        dx = g[sl(6)] - g[sl(0)]
        dy = g[sl(7)] - g[sl(1)]
        o[osl(0)] = dx * dx
        o[osl(1)] = dx * dy
        o[osl(2)] = dy * dy
        for ch in range(4):
          du = g[sl(8 + ch)] - g[sl(2 + ch)]
          o[osl(3 + 2 * ch)] = dx * du
          o[osl(4 + 2 * ch)] = dy * du

    def fire_scatters(b, j, buf):
      ob = buf * 11 * _CHUNK
      rj = ridx.at[b, j]
      return [
          pltpu.async_copy(
              o.at[pl.ds(ob + t * _CHUNK, _CHUNK)], accs[t].at[rj], sems,
              add=True)
          for t in range(11)
      ]

    def process_super(b):
      gh = fire_gathers(b, 0, 0)
      pend = [None] * _NBUF
      for j in range(_CHUNKS_PER_SUPER):
        buf = j % _NBUF
        gh_next = (fire_gathers(b, j + 1, (j + 1) % _NBUF)
                   if j + 1 < _CHUNKS_PER_SUPER else None)
        for h in gh:
          h.wait()
        if pend[buf] is not None:
          for h in pend[buf]:
            h.wait()
        compute(buf)
        pend[buf] = fire_scatters(b, j, buf)
        gh = gh_next
      for p in pend:
        if p is not None:
          for h in p:
            h.wait()

    pltpu.sync_copy(row_hbm.at[pl.ds(tile_base_blk, _CHUNKS_PER_SUPER)],
                    ridx.at[0])
    pltpu.sync_copy(col_hbm.at[pl.ds(tile_base_blk, _CHUNKS_PER_SUPER)],
                    cidx.at[0])

    def pair_body(k, carry):
      blk0 = tile_base_blk + k * 2 * _CHUNKS_PER_SUPER
      h1 = fire_idx(blk0 + _CHUNKS_PER_SUPER, 1)
      process_super(0)
      for h in h1:
        h.wait()
      blk2 = lax.min(blk0 + 2 * _CHUNKS_PER_SUPER,
                     nblks - _CHUNKS_PER_SUPER)
      h2 = fire_idx(blk2, 0)
      process_super(1)
      for h in h2:
        h.wait()
      return carry

    lax.fori_loop(0, pairs_per_tile, pair_body, 0)

    plsc.subcore_barrier()
    for t in range(11):
      off = (cid * 11 + t) * npad + sid * node_chunk
      pltpu.sync_copy(accs[t].at[nsl], out_hbm.at[pl.ds(off, node_chunk)])

  return kern(row2d, col2d, px, py, f0, f1, f2, f3, zeros)


def _solve_body(acc_ref, out_ref):
  a = acc_ref[...]
  m = a[0] + a[1]
  mxx = m[0] + _EPS
  mxy = m[1]
  myy = m[2] + _EPS
  inv_det = 1.0 / (mxx * myy - mxy * mxy)
  for ch in range(4):
    vx = m[3 + 2 * ch]
    vy = m[4 + 2 * ch]
    gx = (myy * vx - mxy * vy) * inv_det
    gy = (mxx * vy - mxy * vx) * inv_det
    out_ref[ch, 0, :] = jnp.clip(gx, -_GRAD_LIMIT, _GRAD_LIMIT)
    out_ref[ch, 1, :] = jnp.clip(gy, -_GRAD_LIMIT, _GRAD_LIMIT)


def kernel(pos, edge_index, field):
  n = pos.shape[0]
  e = edge_index.shape[1]
  c = field.shape[1]
  del c

  npad = ((n + 2047) // 2048) * 2048

  per_tile_unit = 2 * _SUPER * _NUM_TILES
  epad = ((e + per_tile_unit - 1) // per_tile_unit) * per_tile_unit
  pairs_per_tile = epad // per_tile_unit

  row = edge_index[0]
  col = edge_index[1]
  pad_e = epad - e
  dummy = jnp.full((pad_e,), n, dtype=jnp.int32)
  row_p = jnp.concatenate([row, dummy]).reshape(epad // _CHUNK, _CHUNK)
  col_p = jnp.concatenate([col, dummy]).reshape(epad // _CHUNK, _CHUNK)

  pad_n = npad - n
  tblpad = lambda v: jnp.concatenate(
      [v, jnp.zeros((pad_n,), jnp.float32)])
  px = tblpad(pos[:, 0])
  py = tblpad(pos[:, 1])
  f0 = tblpad(field[:, 0])
  f1 = tblpad(field[:, 1])
  f2 = tblpad(field[:, 2])
  f3 = tblpad(field[:, 3])
  zeros = jnp.zeros((npad,), jnp.float32)

  acc = _sc_accumulate(row_p, col_p, px, py, f0, f1, f2, f3, zeros, npad,
                       pairs_per_tile)
  acc = acc.reshape(_NUM_CORES, 11, npad)

  bs = npad // 16
  grads = pl.pallas_call(
      _solve_body,
      grid=(npad // bs,),
      in_specs=[pl.BlockSpec((_NUM_CORES, 11, bs), lambda i: (0, 0, i))],
      out_specs=pl.BlockSpec((4, 2, bs), lambda i: (0, 0, i)),
      out_shape=jax.ShapeDtypeStruct((4, 2, npad), jnp.float32),
  )(acc)

  return jnp.transpose(grads[:, :, :n], (0, 2, 1))


# speedup vs baseline: 1.3991x; 1.3991x over previous
import functools

import jax
import jax.numpy as jnp
from jax import lax
from jax.experimental import pallas as pl
from jax.experimental.pallas import tpu as pltpu
import jax.experimental.pallas.tpu_sc as plsc

_GRAD_LIMIT = 30000.0
_EPS = 1e-08

_NUM_CORES = 2
_NUM_SUBCORES = 16
_NUM_TILES = _NUM_CORES * _NUM_SUBCORES

_CHUNK = 128
_CHUNKS_PER_SUPER = 8
_SUPER = _CHUNK * _CHUNKS_PER_SUPER
_NBUF = 2


def _sc_accumulate(row2d, col2d, px, py, f0, f1, f2, f3, zeros, npad,
                   pairs_per_tile):
  per_tile = pairs_per_tile * 2 * _SUPER
  node_chunk = npad // _NUM_SUBCORES

  mesh = plsc.VectorSubcoreMesh(core_axis_name="c", subcore_axis_name="s")

  tbl_t = [pltpu.VMEM_SHARED((npad,), jnp.float32) for _ in range(6)]
  acc_t = [pltpu.VMEM_SHARED((npad,), jnp.float32) for _ in range(11)]

  nblks = row2d.shape[0]

  @functools.partial(
      pl.kernel,
      out_type=jax.ShapeDtypeStruct((_NUM_CORES * 11 * npad,), jnp.float32),
      mesh=mesh,
      scratch_types=tbl_t + acc_t + [
          pltpu.VMEM((2, _CHUNKS_PER_SUPER, _CHUNK), jnp.int32),
          pltpu.VMEM((2, _CHUNKS_PER_SUPER, _CHUNK), jnp.int32),
          pltpu.VMEM((_NBUF * 12 * _CHUNK,), jnp.float32),
          pltpu.VMEM((_NBUF * 11 * _CHUNK,), jnp.float32),
          pltpu.SemaphoreType.DMA,
          pltpu.SemaphoreType.DMA,
          pltpu.SemaphoreType.DMA,
      ],
  )
  def kern(row_hbm, col_hbm, px_h, py_h, f0_h, f1_h, f2_h, f3_h, z_h,
           out_hbm, sx, sy, sf0, sf1, sf2, sf3,
           a0, a1, a2, a3, a4, a5, a6, a7, a8, a9, a10,
           ridx, cidx, g, o, semg, sems, semi):
    cid = lax.axis_index("c")
    sid = lax.axis_index("s")
    tbls = (sx, sy, sf0, sf1, sf2, sf3)
    tbl_srcs = (px_h, py_h, f0_h, f1_h, f2_h, f3_h)
    accs = (a0, a1, a2, a3, a4, a5, a6, a7, a8, a9, a10)

    nsl = pl.ds(sid * node_chunk, node_chunk)
    for dst, src in zip(tbls, tbl_srcs):
      pltpu.sync_copy(src.at[nsl], dst.at[nsl])
    for a in accs:
      pltpu.sync_copy(z_h.at[nsl], a.at[nsl])
    plsc.subcore_barrier()

    tile_base_blk = (cid * _NUM_SUBCORES + sid) * (per_tile // _CHUNK)

    def fire_idx(blk, b):
      return [
          pltpu.async_copy(
              row_hbm.at[pl.ds(blk, _CHUNKS_PER_SUPER)], ridx.at[b], semi),
          pltpu.async_copy(
              col_hbm.at[pl.ds(blk, _CHUNKS_PER_SUPER)], cidx.at[b], semi),
      ]

    def fire_gathers(b, j, buf):
      gb = buf * 12 * _CHUNK
      rj = ridx.at[b, j]
      cj = cidx.at[b, j]
      hs = []
      for t in range(6):
        hs.append(pltpu.async_copy(
            tbls[t].at[rj], g.at[pl.ds(gb + t * _CHUNK, _CHUNK)], semg))
        hs.append(pltpu.async_copy(
            tbls[t].at[cj], g.at[pl.ds(gb + (6 + t) * _CHUNK, _CHUNK)],
            semg))
      return hs

    def compute(buf):
      gb = buf * 12 * _CHUNK
      ob = buf * 11 * _CHUNK
      for i in range(_CHUNK // 16):
        sl = lambda t: pl.ds(gb + t * _CHUNK + i * 16, 16)
        osl = lambda t: pl.ds(ob + t * _CHUNK + i * 16, 16)
        dx = g[sl(6)] - g[sl(0)]
        dy = g[sl(7)] - g[sl(1)]
        o[osl(0)] = dx * dx
        o[osl(1)] = dx * dy
        o[osl(2)] = dy * dy
        for ch in range(4):
          du = g[sl(8 + ch)] - g[sl(2 + ch)]
          o[osl(3 + 2 * ch)] = dx * du
          o[osl(4 + 2 * ch)] = dy * du

    def fire_scatters(b, j, buf):
      ob = buf * 11 * _CHUNK
      rj = ridx.at[b, j]
      return [
          pltpu.async_copy(
              o.at[pl.ds(ob + t * _CHUNK, _CHUNK)], accs[t].at[rj], sems,
              add=True)
          for t in range(11)
      ]

    def process_super(b):
      gh = fire_gathers(b, 0, 0)
      pend = [None] * _NBUF
      for j in range(_CHUNKS_PER_SUPER):
        buf = j % _NBUF
        gh_next = (fire_gathers(b, j + 1, (j + 1) % _NBUF)
                   if j + 1 < _CHUNKS_PER_SUPER else None)
        for h in gh:
          h.wait()
        if pend[buf] is not None:
          for h in pend[buf]:
            h.wait()
        compute(buf)
        pend[buf] = fire_scatters(b, j, buf)
        gh = gh_next
      for p in pend:
        if p is not None:
          for h in p:
            h.wait()

    pltpu.sync_copy(row_hbm.at[pl.ds(tile_base_blk, _CHUNKS_PER_SUPER)],
                    ridx.at[0])
    pltpu.sync_copy(col_hbm.at[pl.ds(tile_base_blk, _CHUNKS_PER_SUPER)],
                    cidx.at[0])

    def pair_body(k, carry):
      blk0 = tile_base_blk + k * 2 * _CHUNKS_PER_SUPER
      h1 = fire_idx(blk0 + _CHUNKS_PER_SUPER, 1)
      process_super(0)
      for h in h1:
        h.wait()
      blk2 = lax.min(blk0 + 2 * _CHUNKS_PER_SUPER,
                     nblks - _CHUNKS_PER_SUPER)
      h2 = fire_idx(blk2, 0)
      process_super(1)
      for h in h2:
        h.wait()
      return carry

    lax.fori_loop(0, pairs_per_tile, pair_body, 0)

    plsc.subcore_barrier()
    for t in range(11):
      off = (cid * 11 + t) * npad + sid * node_chunk
      pltpu.sync_copy(accs[t].at[nsl], out_hbm.at[pl.ds(off, node_chunk)])

  return kern(row2d, col2d, px, py, f0, f1, f2, f3, zeros)


def _solve_body(acc_ref, out_ref):
  a = acc_ref[...]
  m = a[0] + a[1]
  mxx = m[0] + _EPS
  mxy = m[1]
  myy = m[2] + _EPS
  inv_det = 1.0 / (mxx * myy - mxy * mxy)
  for ch in range(4):
    vx = m[3 + 2 * ch]
    vy = m[4 + 2 * ch]
    gx = (myy * vx - mxy * vy) * inv_det
    gy = (mxx * vy - mxy * vx) * inv_det
    out_ref[ch, 0, :] = jnp.clip(gx, -_GRAD_LIMIT, _GRAD_LIMIT)
    out_ref[ch, 1, :] = jnp.clip(gy, -_GRAD_LIMIT, _GRAD_LIMIT)


def kernel(pos, edge_index, field):
  n = pos.shape[0]
  e = edge_index.shape[1]
  c = field.shape[1]
  del c

  npad = ((n + 2047) // 2048) * 2048

  per_tile_unit = 2 * _SUPER * _NUM_TILES
  epad = ((e + per_tile_unit - 1) // per_tile_unit) * per_tile_unit
  pairs_per_tile = epad // per_tile_unit

  row = edge_index[0]
  col = edge_index[1]
  pad_e = epad - e
  dummy = jnp.full((pad_e,), n, dtype=jnp.int32)
  row_p = jnp.concatenate([row, dummy]).reshape(epad // _CHUNK, _CHUNK)
  col_p = jnp.concatenate([col, dummy]).reshape(epad // _CHUNK, _CHUNK)

  pad_n = npad - n
  tblpad = lambda v: jnp.concatenate(
      [v, jnp.zeros((pad_n,), jnp.float32)])
  px = tblpad(pos[:, 0])
  py = tblpad(pos[:, 1])
  f0 = tblpad(field[:, 0])
  f1 = tblpad(field[:, 1])
  f2 = tblpad(field[:, 2])
  f3 = tblpad(field[:, 3])
  zeros = jnp.zeros((npad,), jnp.float32)

  acc = _sc_accumulate(row_p, col_p, px, py, f0, f1, f2, f3, zeros, npad,
                       pairs_per_tile)
  acc = acc.reshape(_NUM_CORES, 11, npad)

  bs = npad // 16
  grads = pl.pallas_call(
      _solve_body,
      grid=(npad // bs,),
      in_specs=[pl.BlockSpec((_NUM_CORES, 11, bs), lambda i: (0, 0, i))],
      out_specs=pl.BlockSpec((4, 2, bs), lambda i: (0, 0, i)),
      out_shape=jax.ShapeDtypeStruct((4, 2, npad), jnp.float32),
  )(acc)

  return jnp.transpose(grads[:, :, :n], (0, 2, 1))


# speedup vs baseline: 1.4000x; 1.0006x over previous
import functools

import jax
import jax.numpy as jnp
from jax import lax
from jax.experimental import pallas as pl
from jax.experimental.pallas import tpu as pltpu
import jax.experimental.pallas.tpu_sc as plsc

_GRAD_LIMIT = 30000.0
_EPS = 1e-08

_NUM_CORES = 2
_NUM_SUBCORES = 16
_NUM_TILES = _NUM_CORES * _NUM_SUBCORES

_CHUNK = 128
_CHUNKS_PER_SUPER = 8
_SUPER = _CHUNK * _CHUNKS_PER_SUPER
_NBUF = 3


def _sc_accumulate(row2d, col2d, px, py, f0, f1, f2, f3, zeros, npad,
                   pairs_per_tile):
  per_tile = pairs_per_tile * 2 * _SUPER
  node_chunk = npad // _NUM_SUBCORES

  mesh = plsc.VectorSubcoreMesh(core_axis_name="c", subcore_axis_name="s")

  tbl_t = [pltpu.VMEM_SHARED((npad,), jnp.float32) for _ in range(6)]
  acc_t = [pltpu.VMEM_SHARED((npad,), jnp.float32) for _ in range(11)]

  nblks = row2d.shape[0]

  @functools.partial(
      pl.kernel,
      out_type=jax.ShapeDtypeStruct((_NUM_CORES * 11 * npad,), jnp.float32),
      mesh=mesh,
      scratch_types=tbl_t + acc_t + [
          pltpu.VMEM((2, _CHUNKS_PER_SUPER, _CHUNK), jnp.int32),
          pltpu.VMEM((2, _CHUNKS_PER_SUPER, _CHUNK), jnp.int32),
          pltpu.VMEM((_NBUF * 12 * _CHUNK,), jnp.float32),
          pltpu.VMEM((_NBUF * 11 * _CHUNK,), jnp.float32),
          pltpu.SemaphoreType.DMA,
          pltpu.SemaphoreType.DMA,
          pltpu.SemaphoreType.DMA,
      ],
  )
  def kern(row_hbm, col_hbm, px_h, py_h, f0_h, f1_h, f2_h, f3_h, z_h,
           out_hbm, sx, sy, sf0, sf1, sf2, sf3,
           a0, a1, a2, a3, a4, a5, a6, a7, a8, a9, a10,
           ridx, cidx, g, o, semg, sems, semi):
    cid = lax.axis_index("c")
    sid = lax.axis_index("s")
    tbls = (sx, sy, sf0, sf1, sf2, sf3)
    tbl_srcs = (px_h, py_h, f0_h, f1_h, f2_h, f3_h)
    accs = (a0, a1, a2, a3, a4, a5, a6, a7, a8, a9, a10)

    nsl = pl.ds(sid * node_chunk, node_chunk)
    for dst, src in zip(tbls, tbl_srcs):
      pltpu.sync_copy(src.at[nsl], dst.at[nsl])
    for a in accs:
      pltpu.sync_copy(z_h.at[nsl], a.at[nsl])
    plsc.subcore_barrier()

    tile_base_blk = (cid * _NUM_SUBCORES + sid) * (per_tile // _CHUNK)

    def fire_idx(blk, b):
      return [
          pltpu.async_copy(
              row_hbm.at[pl.ds(blk, _CHUNKS_PER_SUPER)], ridx.at[b], semi),
          pltpu.async_copy(
              col_hbm.at[pl.ds(blk, _CHUNKS_PER_SUPER)], cidx.at[b], semi),
      ]

    def fire_gathers(b, j, buf):
      gb = buf * 12 * _CHUNK
      rj = ridx.at[b, j]
      cj = cidx.at[b, j]
      hs = []
      for t in range(6):
        hs.append(pltpu.async_copy(
            tbls[t].at[rj], g.at[pl.ds(gb + t * _CHUNK, _CHUNK)], semg))
        hs.append(pltpu.async_copy(
            tbls[t].at[cj], g.at[pl.ds(gb + (6 + t) * _CHUNK, _CHUNK)],
            semg))
      return hs

    def compute(buf):
      gb = buf * 12 * _CHUNK
      ob = buf * 11 * _CHUNK
      for i in range(_CHUNK // 16):
        sl = lambda t: pl.ds(gb + t * _CHUNK + i * 16, 16)
        osl = lambda t: pl.ds(ob + t * _CHUNK + i * 16, 16)
        dx = g[sl(6)] - g[sl(0)]
        dy = g[sl(7)] - g[sl(1)]
        o[osl(0)] = dx * dx
        o[osl(1)] = dx * dy
        o[osl(2)] = dy * dy
        for ch in range(4):
          du = g[sl(8 + ch)] - g[sl(2 + ch)]
          o[osl(3 + 2 * ch)] = dx * du
          o[osl(4 + 2 * ch)] = dy * du

    def fire_scatters(b, j, buf):
      ob = buf * 11 * _CHUNK
      rj = ridx.at[b, j]
      return [
          pltpu.async_copy(
              o.at[pl.ds(ob + t * _CHUNK, _CHUNK)], accs[t].at[rj], sems,
              add=True)
          for t in range(11)
      ]

    def process_super(b):
      gh = fire_gathers(b, 0, 0)
      pend = [None] * _NBUF
      for j in range(_CHUNKS_PER_SUPER):
        buf = j % _NBUF
        gh_next = (fire_gathers(b, j + 1, (j + 1) % _NBUF)
                   if j + 1 < _CHUNKS_PER_SUPER else None)
        for h in gh:
          h.wait()
        if pend[buf] is not None:
          for h in pend[buf]:
            h.wait()
        compute(buf)
        pend[buf] = fire_scatters(b, j, buf)
        gh = gh_next
      for p in pend:
        if p is not None:
          for h in p:
            h.wait()

    pltpu.sync_copy(row_hbm.at[pl.ds(tile_base_blk, _CHUNKS_PER_SUPER)],
                    ridx.at[0])
    pltpu.sync_copy(col_hbm.at[pl.ds(tile_base_blk, _CHUNKS_PER_SUPER)],
                    cidx.at[0])

    def pair_body(k, carry):
      blk0 = tile_base_blk + k * 2 * _CHUNKS_PER_SUPER
      h1 = fire_idx(blk0 + _CHUNKS_PER_SUPER, 1)
      process_super(0)
      for h in h1:
        h.wait()
      blk2 = lax.min(blk0 + 2 * _CHUNKS_PER_SUPER,
                     nblks - _CHUNKS_PER_SUPER)
      h2 = fire_idx(blk2, 0)
      process_super(1)
      for h in h2:
        h.wait()
      return carry

    lax.fori_loop(0, pairs_per_tile, pair_body, 0)

    plsc.subcore_barrier()
    for t in range(11):
      off = (cid * 11 + t) * npad + sid * node_chunk
      pltpu.sync_copy(accs[t].at[nsl], out_hbm.at[pl.ds(off, node_chunk)])

  return kern(row2d, col2d, px, py, f0, f1, f2, f3, zeros)


def _solve_body(acc_ref, out_ref):
  a = acc_ref[...]
  m = a[0] + a[1]
  mxx = m[0] + _EPS
  mxy = m[1]
  myy = m[2] + _EPS
  inv_det = 1.0 / (mxx * myy - mxy * mxy)
  for ch in range(4):
    vx = m[3 + 2 * ch]
    vy = m[4 + 2 * ch]
    gx = (myy * vx - mxy * vy) * inv_det
    gy = (mxx * vy - mxy * vx) * inv_det
    out_ref[ch, 0, :] = jnp.clip(gx, -_GRAD_LIMIT, _GRAD_LIMIT)
    out_ref[ch, 1, :] = jnp.clip(gy, -_GRAD_LIMIT, _GRAD_LIMIT)


def kernel(pos, edge_index, field):
  n = pos.shape[0]
  e = edge_index.shape[1]
  c = field.shape[1]
  del c

  npad = ((n + 2047) // 2048) * 2048

  per_tile_unit = 2 * _SUPER * _NUM_TILES
  epad = ((e + per_tile_unit - 1) // per_tile_unit) * per_tile_unit
  pairs_per_tile = epad // per_tile_unit

  row = edge_index[0]
  col = edge_index[1]
  pad_e = epad - e
  dummy = jnp.full((pad_e,), n, dtype=jnp.int32)
  row_p = jnp.concatenate([row, dummy]).reshape(epad // _CHUNK, _CHUNK)
  col_p = jnp.concatenate([col, dummy]).reshape(epad // _CHUNK, _CHUNK)

  pad_n = npad - n
  tblpad = lambda v: jnp.concatenate(
      [v, jnp.zeros((pad_n,), jnp.float32)])
  px = tblpad(pos[:, 0])
  py = tblpad(pos[:, 1])
  f0 = tblpad(field[:, 0])
  f1 = tblpad(field[:, 1])
  f2 = tblpad(field[:, 2])
  f3 = tblpad(field[:, 3])
  zeros = jnp.zeros((npad,), jnp.float32)

  acc = _sc_accumulate(row_p, col_p, px, py, f0, f1, f2, f3, zeros, npad,
                       pairs_per_tile)
  acc = acc.reshape(_NUM_CORES, 11, npad)

  bs = npad // 16
  grads = pl.pallas_call(
      _solve_body,
      grid=(npad // bs,),
      in_specs=[pl.BlockSpec((_NUM_CORES, 11, bs), lambda i: (0, 0, i))],
      out_specs=pl.BlockSpec((4, 2, bs), lambda i: (0, 0, i)),
      out_shape=jax.ShapeDtypeStruct((4, 2, npad), jnp.float32),
  )(acc)

  return jnp.transpose(grads[:, :, :n], (0, 2, 1))
